# baseline (speedup 1.0000x reference)
import jax
import jax.numpy as jnp
from jax import lax
from jax.experimental import pallas as pl
from jax.experimental.pallas import tpu as pltpu
from jax.experimental.pallas import tpu_sc as plsc

_NC = 2
_NS = 16
_NW = _NC * _NS

_E = 320000
_D = 128
_N = 10000
_NP = 10240
_RPT = _NP // _NS

_C = 80
_S = 3
_CS = 80
_SS = 4

_ES0 = 163840
_ES1 = _E - _ES0

_sc_mesh = plsc.VectorSubcoreMesh(core_axis_name="c", subcore_axis_name="s")


def _make_gather(es):
    epw = es // _NW
    nchunk = epw // _C

    def body(hid_hbm, rela_hbm, qemb_hbm, idxc_hbm, rows_out,
             idx_v, rows_v, gsem, wsem):
        wid = lax.axis_index("s") * _NC + lax.axis_index("c")
        base0 = wid * epw
        tabs = (hid_hbm, rela_hbm, qemb_hbm)
        iw = nchunk * 3 * _C
        pltpu.sync_copy(idxc_hbm.at[pl.ds(wid * iw, iw)], idx_v)

        def step(k, carry):
            b = base0 + k * (_S * _C)
            cps = [pltpu.async_copy(
                tabs[t].at[idx_v.at[pl.ds((k * _S + s) * 3 * _C + t * _C, _C)]],
                rows_v.at[s, t], gsem)
                for s in range(_S) for t in range(3)]
            for cp in cps:
                cp.wait()
            cps = [pltpu.async_copy(rows_v.at[s, t],
                                    rows_out.at[t, pl.ds(b + s * _C, _C)], wsem)
                   for s in range(_S) for t in range(3)]
            for cp in cps:
                cp.wait()
            return carry

        def chunk_tail(i, carry):
            b = base0 + i * _C
            cps = [pltpu.async_copy(
                tabs[t].at[idx_v.at[pl.ds(i * 3 * _C + t * _C, _C)]],
                rows_v.at[0, t], gsem) for t in range(3)]
            for cp in cps:
                cp.wait()
            cps = [pltpu.async_copy(rows_v.at[0, t],
                                    rows_out.at[t, pl.ds(b, _C)], wsem)
                   for t in range(3)]
            for cp in cps:
                cp.wait()
            return carry

        lax.fori_loop(0, nchunk // _S, step, 0)
        lax.fori_loop((nchunk // _S) * _S, nchunk, chunk_tail, 0)

    return pl.kernel(
        body,
        out_type=jax.ShapeDtypeStruct((3, es, _D), jnp.float32),
        mesh=_sc_mesh,
        scratch_types=[
            pltpu.VMEM((nchunk * 3 * _C,), jnp.int32),
            pltpu.VMEM((_S, 3, _C, _D), jnp.float32),
            pltpu.SemaphoreType.DMA,
            pltpu.SemaphoreType.DMA,
        ],
    )


_gather_s0 = _make_gather(_ES0)
_gather_s1 = _make_gather(_ES1)


def _make_scatter(es):
    nchs = es // _CS
    full = nchs // _NW
    rem = nchs % _NW

    def body(msg_hbm, obj2d_hbm, zeros_hbm, out_hbm,
             obj_v, msg_v, acc_sh, isem, lsem, ssem):
        cid = lax.axis_index("c")
        sid = lax.axis_index("s")
        wid = sid * _NC + cid
        row0 = sid * _RPT
        trip = full + jnp.where(wid < rem, 1, 0)

        pltpu.sync_copy(zeros_hbm.at[pl.ds(row0, _RPT)],
                        acc_sh.at[pl.ds(row0, _RPT)])
        plsc.subcore_barrier()

        def chunk_tail(j, carry):
            cidx = wid + j * _NW
            pltpu.sync_copy(obj2d_hbm.at[cidx], obj_v.at[0])
            pltpu.sync_copy(msg_hbm.at[pl.ds(cidx * _CS, _CS)], msg_v.at[0])
            pltpu.sync_copy(msg_v.at[0], acc_sh.at[obj_v.at[0]], add=True)
            return carry

        def step(k, carry):
            cps = []
            for s in range(_SS):
                cidx = wid + (k * _SS + s) * _NW
                cps.append(pltpu.async_copy(obj2d_hbm.at[cidx], obj_v.at[s], isem))
                cps.append(pltpu.async_copy(msg_hbm.at[pl.ds(cidx * _CS, _CS)],
                                            msg_v.at[s], lsem))
            for cp in cps:
                cp.wait()
            cps = [pltpu.async_copy(msg_v.at[s], acc_sh.at[obj_v.at[s]], ssem,
                                    add=True) for s in range(_SS)]
            for cp in cps:
                cp.wait()
            return carry

        lax.fori_loop(0, full // _SS, step, 0)
        lax.fori_loop((full // _SS) * _SS, trip, chunk_tail, 0)
        plsc.subcore_barrier()
        pltpu.sync_copy(acc_sh.at[pl.ds(row0, _RPT)],
                        out_hbm.at[pl.ds(cid * _NP + row0, _RPT)])

    return pl.kernel(
        body,
        out_type=jax.ShapeDtypeStruct((_NC * _NP, _D), jnp.float32),
        mesh=_sc_mesh,
        scratch_types=[
            pltpu.VMEM((_SS, _CS), jnp.int32),
            pltpu.VMEM((_SS, _CS, _D), jnp.float32),
            pltpu.VMEM_SHARED((_NP, _D), jnp.float32),
            pltpu.SemaphoreType.DMA,
            pltpu.SemaphoreType.DMA,
            pltpu.SemaphoreType.DMA,
        ],
    )


_scatter_s0 = _make_scatter(_ES0)
_scatter_s1 = _make_scatter(_ES1)


_BE = 2048


def _dense_body(rows_ref, Ws_ref, Wsb_ref, Wr_ref, Wq_ref,
                Wqr_ref, wa_ref, wab_ref, alpha_ref, msg_ref):
    hs = rows_ref[0]
    hr = rows_ref[1]
    hq = rows_ref[2]
    pre = (jnp.dot(hs, Ws_ref[...], preferred_element_type=jnp.float32)
           + jnp.dot(hr, Wr_ref[...], preferred_element_type=jnp.float32)
           + jnp.dot(hq, Wq_ref[...], preferred_element_type=jnp.float32)
           + jnp.dot(hr * hq, Wqr_ref[...], preferred_element_type=jnp.float32)
           + Wsb_ref[...])
    pre = jnp.maximum(pre, 0.0)
    z = jnp.dot(pre, wa_ref[...], preferred_element_type=jnp.float32) + wab_ref[...]
    av = jax.nn.sigmoid(z)
    alpha_ref[...] = av
    msg_ref[...] = av * (hs * hr)


def _dense_stage(rows, Ws_W, Ws_b, Wr_W, Wq_W, Wqr_W, wa_W, wa_b):
    es = rows.shape[1]
    a = Ws_W.shape[1]
    full = lambda s: pl.BlockSpec(s, lambda i: (0,) * len(s))
    return pl.pallas_call(
        _dense_body,
        out_shape=(jax.ShapeDtypeStruct((es, 1), jnp.float32),
                   jax.ShapeDtypeStruct((es, _D), jnp.float32)),
        grid=(pl.cdiv(es, _BE),),
        in_specs=[
            pl.BlockSpec((3, _BE, _D), lambda i: (0, i, 0)),
            full((_D, a)), full((1, a)), full((_D, a)), full((_D, a)),
            full((_D, a)), full((a, 1)), full((1, 1)),
        ],
        out_specs=(pl.BlockSpec((_BE, 1), lambda i: (i, 0)),
                   pl.BlockSpec((_BE, _D), lambda i: (i, 0))),
    )(rows, Ws_W, Ws_b.reshape(1, a), Wr_W, Wq_W, Wqr_W, wa_W,
      wa_b.reshape(1, 1))


def _final_body(p_ref, w_ref, o_ref):
    s = p_ref[0, 0]
    for k in range(1, 4):
        s = s + p_ref[k // 2, k % 2]
    o_ref[...] = jnp.dot(s, w_ref[...], preferred_element_type=jnp.float32)


def _final_stage(parts, Wh_W):
    bn = 1000
    return pl.pallas_call(
        _final_body,
        out_shape=jax.ShapeDtypeStruct((_N, _D), jnp.float32),
        grid=(_N // bn,),
        in_specs=[
            pl.BlockSpec((2, 2, bn, _D), lambda i: (0, 0, i, 0)),
            pl.BlockSpec((_D, _D), lambda i: (0, 0)),
        ],
        out_specs=pl.BlockSpec((bn, _D), lambda i: (i, 0)),
    )(parts, Wh_W)


def kernel(q_sub, q_rel, q_emb, rela_embed, hidden, edges, nodes,
           old_nodes_new_idx, Ws_W, Ws_b, Wr_W, Wq_W, Wqr_W, wa_W, wa_b, Wh_W):
    l1 = edges.shape[0]
    n1 = nodes.shape[0]
    ei = edges.astype(jnp.int32)
    obj = ei[:, 5]
    idx3 = jnp.stack([ei[:, 4], ei[:, 2], ei[:, 0]], axis=0)
    idxc = idx3.reshape(3, _E // _C, _C).transpose(1, 0, 2)
    zeros = jnp.zeros((_NP, _D), jnp.float32)
    obj2d = obj.reshape(_E // _CS, _CS)

    g0 = _ES0 // _C
    c0 = _ES0 // _CS
    idxc0 = idxc[:g0].reshape(-1)
    idxc1 = idxc[g0:].reshape(-1)

    rows0 = _gather_s0(hidden, rela_embed, q_emb, idxc0)
    alpha0, msg0 = _dense_stage(rows0, Ws_W, Ws_b, Wr_W, Wq_W, Wqr_W, wa_W, wa_b)
    rows1 = _gather_s1(hidden, rela_embed, q_emb, idxc1)
    alpha1, msg1 = _dense_stage(rows1, Ws_W, Ws_b, Wr_W, Wq_W, Wqr_W, wa_W, wa_b)
    parts0 = _scatter_s0(msg0, obj2d[:c0], zeros)
    parts1 = _scatter_s1(msg1, obj2d[c0:], zeros)

    alpha = jnp.concatenate([alpha0, alpha1], axis=0)
    parts = jnp.stack([parts0.reshape(_NC, _NP, _D),
                       parts1.reshape(_NC, _NP, _D)], axis=0)
    hidden_new = _final_stage(parts, Wh_W)

    nq = q_sub.shape[0]
    num_node = jnp.array([n1 * 1.0 / nq, n1 * 1.0 / nq], dtype=jnp.float32)
    num_edge = jnp.array([l1 * 1.0 / nq, l1 * 1.0 / nq], dtype=jnp.float32)
    return (num_node, num_edge, hidden_new, alpha, nodes, edges,
            old_nodes_new_idx)


# speedup vs baseline: 1.0136x; 1.0136x over previous
import jax
import jax.numpy as jnp
from jax import lax
from jax.experimental import pallas as pl
from jax.experimental.pallas import tpu as pltpu
from jax.experimental.pallas import tpu_sc as plsc

_NC = 2
_NS = 16
_NW = _NC * _NS

_E = 320000
_D = 128
_N = 10000
_NP = 10240
_RPT = _NP // _NS

_C = 80
_S = 3
_CS = 128
_SS = 2

_ES0 = 163840
_ES1 = _E - _ES0

_sc_mesh = plsc.VectorSubcoreMesh(core_axis_name="c", subcore_axis_name="s")


def _make_gather(es):
    epw = es // _NW
    nchunk = epw // _C

    def body(hid_hbm, rela_hbm, qemb_hbm, idxc_hbm, rows_out,
             idx_v, rows_v, gsem, wsem):
        wid = lax.axis_index("s") * _NC + lax.axis_index("c")
        base0 = wid * epw
        tabs = (hid_hbm, rela_hbm, qemb_hbm)
        iw = nchunk * 3 * _C
        pltpu.sync_copy(idxc_hbm.at[pl.ds(wid * iw, iw)], idx_v)

        def step(k, carry):
            b = base0 + k * (_S * _C)
            cps = [pltpu.async_copy(
                tabs[t].at[idx_v.at[pl.ds((k * _S + s) * 3 * _C + t * _C, _C)]],
                rows_v.at[s, t], gsem)
                for s in range(_S) for t in range(3)]
            for cp in cps:
                cp.wait()
            cps = [pltpu.async_copy(rows_v.at[s, t],
                                    rows_out.at[t, pl.ds(b + s * _C, _C)], wsem)
                   for s in range(_S) for t in range(3)]
            for cp in cps:
                cp.wait()
            return carry

        def chunk_tail(i, carry):
            b = base0 + i * _C
            cps = [pltpu.async_copy(
                tabs[t].at[idx_v.at[pl.ds(i * 3 * _C + t * _C, _C)]],
                rows_v.at[0, t], gsem) for t in range(3)]
            for cp in cps:
                cp.wait()
            cps = [pltpu.async_copy(rows_v.at[0, t],
                                    rows_out.at[t, pl.ds(b, _C)], wsem)
                   for t in range(3)]
            for cp in cps:
                cp.wait()
            return carry

        lax.fori_loop(0, nchunk // _S, step, 0)
        lax.fori_loop((nchunk // _S) * _S, nchunk, chunk_tail, 0)

    return pl.kernel(
        body,
        out_type=jax.ShapeDtypeStruct((3, es, _D), jnp.float32),
        mesh=_sc_mesh,
        scratch_types=[
            pltpu.VMEM((nchunk * 3 * _C,), jnp.int32),
            pltpu.VMEM((_S, 3, _C, _D), jnp.float32),
            pltpu.SemaphoreType.DMA,
            pltpu.SemaphoreType.DMA,
        ],
    )


_gather_s0 = _make_gather(_ES0)
_gather_s1 = _make_gather(_ES1)


def _make_scatter(es):
    nchs = es // _CS
    full = nchs // _NW
    rem = nchs % _NW

    def body(msg_hbm, obj2d_hbm, zeros_hbm, out_hbm,
             obj_v, msg_v, acc_sh, isem, lsem, ssem):
        cid = lax.axis_index("c")
        sid = lax.axis_index("s")
        wid = sid * _NC + cid
        row0 = sid * _RPT
        trip = full + jnp.where(wid < rem, 1, 0)

        pltpu.sync_copy(zeros_hbm.at[pl.ds(row0, _RPT)],
                        acc_sh.at[pl.ds(row0, _RPT)])
        plsc.subcore_barrier()

        def chunk_tail(j, carry):
            cidx = wid + j * _NW
            pltpu.sync_copy(obj2d_hbm.at[cidx], obj_v.at[0])
            pltpu.sync_copy(msg_hbm.at[pl.ds(cidx * _CS, _CS)], msg_v.at[0])
            pltpu.sync_copy(msg_v.at[0], acc_sh.at[obj_v.at[0]], add=True)
            return carry

        def step(k, carry):
            cps = []
            for s in range(_SS):
                cidx = wid + (k * _SS + s) * _NW
                cps.append(pltpu.async_copy(obj2d_hbm.at[cidx], obj_v.at[s], isem))
                cps.append(pltpu.async_copy(msg_hbm.at[pl.ds(cidx * _CS, _CS)],
                                            msg_v.at[s], lsem))
            for cp in cps:
                cp.wait()
            cps = [pltpu.async_copy(msg_v.at[s], acc_sh.at[obj_v.at[s]], ssem,
                                    add=True) for s in range(_SS)]
            for cp in cps:
                cp.wait()
            return carry

        lax.fori_loop(0, full // _SS, step, 0)
        lax.fori_loop((full // _SS) * _SS, trip, chunk_tail, 0)
        plsc.subcore_barrier()
        pltpu.sync_copy(acc_sh.at[pl.ds(row0, _RPT)],
                        out_hbm.at[pl.ds(cid * _NP + row0, _RPT)])

    return pl.kernel(
        body,
        out_type=jax.ShapeDtypeStruct((_NC * _NP, _D), jnp.float32),
        mesh=_sc_mesh,
        scratch_types=[
            pltpu.VMEM((_SS, _CS), jnp.int32),
            pltpu.VMEM((_SS, _CS, _D), jnp.float32),
            pltpu.VMEM_SHARED((_NP, _D), jnp.float32),
            pltpu.SemaphoreType.DMA,
            pltpu.SemaphoreType.DMA,
            pltpu.SemaphoreType.DMA,
        ],
    )


_scatter_s0 = _make_scatter(_ES0)
_scatter_s1 = _make_scatter(_ES1)


_BE = 4096


def _dense_body(rows_ref, Ws_ref, Wsb_ref, Wr_ref, Wq_ref,
                Wqr_ref, wa_ref, wab_ref, alpha_ref, msg_ref):
    hs = rows_ref[0]
    hr = rows_ref[1]
    hq = rows_ref[2]
    pre = (jnp.dot(hs, Ws_ref[...], preferred_element_type=jnp.float32)
           + jnp.dot(hr, Wr_ref[...], preferred_element_type=jnp.float32)
           + jnp.dot(hq, Wq_ref[...], preferred_element_type=jnp.float32)
           + jnp.dot(hr * hq, Wqr_ref[...], preferred_element_type=jnp.float32)
           + Wsb_ref[...])
    pre = jnp.maximum(pre, 0.0)
    z = jnp.dot(pre, wa_ref[...], preferred_element_type=jnp.float32) + wab_ref[...]
    av = jax.nn.sigmoid(z)
    alpha_ref[...] = av
    msg_ref[...] = av * (hs * hr)


def _dense_stage(rows, Ws_W, Ws_b, Wr_W, Wq_W, Wqr_W, wa_W, wa_b):
    es = rows.shape[1]
    a = Ws_W.shape[1]
    full = lambda s: pl.BlockSpec(s, lambda i: (0,) * len(s))
    return pl.pallas_call(
        _dense_body,
        out_shape=(jax.ShapeDtypeStruct((es, 1), jnp.float32),
                   jax.ShapeDtypeStruct((es, _D), jnp.float32)),
        grid=(pl.cdiv(es, _BE),),
        in_specs=[
            pl.BlockSpec((3, _BE, _D), lambda i: (0, i, 0)),
            full((_D, a)), full((1, a)), full((_D, a)), full((_D, a)),
            full((_D, a)), full((a, 1)), full((1, 1)),
        ],
        out_specs=(pl.BlockSpec((_BE, 1), lambda i: (i, 0)),
                   pl.BlockSpec((_BE, _D), lambda i: (i, 0))),
    )(rows, Ws_W, Ws_b.reshape(1, a), Wr_W, Wq_W, Wqr_W, wa_W,
      wa_b.reshape(1, 1))


def _final_body(p_ref, w_ref, o_ref):
    s = p_ref[0, 0]
    for k in range(1, 4):
        s = s + p_ref[k // 2, k % 2]
    o_ref[...] = jnp.dot(s, w_ref[...], preferred_element_type=jnp.float32)


def _final_stage(parts, Wh_W):
    bn = 1000
    return pl.pallas_call(
        _final_body,
        out_shape=jax.ShapeDtypeStruct((_N, _D), jnp.float32),
        grid=(_N // bn,),
        in_specs=[
            pl.BlockSpec((2, 2, bn, _D), lambda i: (0, 0, i, 0)),
            pl.BlockSpec((_D, _D), lambda i: (0, 0)),
        ],
        out_specs=pl.BlockSpec((bn, _D), lambda i: (i, 0)),
    )(parts, Wh_W)


def kernel(q_sub, q_rel, q_emb, rela_embed, hidden, edges, nodes,
           old_nodes_new_idx, Ws_W, Ws_b, Wr_W, Wq_W, Wqr_W, wa_W, wa_b, Wh_W):
    l1 = edges.shape[0]
    n1 = nodes.shape[0]
    ei = edges.astype(jnp.int32)
    obj = ei[:, 5]
    idx3 = jnp.stack([ei[:, 4], ei[:, 2], ei[:, 0]], axis=0)
    idxc = idx3.reshape(3, _E // _C, _C).transpose(1, 0, 2)
    zeros = jnp.zeros((_NP, _D), jnp.float32)
    obj2d = obj.reshape(_E // _CS, _CS)

    g0 = _ES0 // _C
    c0 = _ES0 // _CS
    idxc0 = idxc[:g0].reshape(-1)
    idxc1 = idxc[g0:].reshape(-1)

    rows0 = _gather_s0(hidden, rela_embed, q_emb, idxc0)
    alpha0, msg0 = _dense_stage(rows0, Ws_W, Ws_b, Wr_W, Wq_W, Wqr_W, wa_W, wa_b)
    rows1 = _gather_s1(hidden, rela_embed, q_emb, idxc1)
    alpha1, msg1 = _dense_stage(rows1, Ws_W, Ws_b, Wr_W, Wq_W, Wqr_W, wa_W, wa_b)
    parts0 = _scatter_s0(msg0, obj2d[:c0], zeros)
    parts1 = _scatter_s1(msg1, obj2d[c0:], zeros)

    alpha = jnp.concatenate([alpha0, alpha1], axis=0)
    parts = jnp.stack([parts0.reshape(_NC, _NP, _D),
                       parts1.reshape(_NC, _NP, _D)], axis=0)
    hidden_new = _final_stage(parts, Wh_W)

    nq = q_sub.shape[0]
    num_node = jnp.array([n1 * 1.0 / nq, n1 * 1.0 / nq], dtype=jnp.float32)
    num_edge = jnp.array([l1 * 1.0 / nq, l1 * 1.0 / nq], dtype=jnp.float32)
    return (num_node, num_edge, hidden_new, alpha, nodes, edges,
            old_nodes_new_idx)


# speedup vs baseline: 1.0281x; 1.0143x over previous
import jax
import jax.numpy as jnp
from jax import lax
from jax.experimental import pallas as pl
from jax.experimental.pallas import tpu as pltpu
from jax.experimental.pallas import tpu_sc as plsc

_NC = 2
_NS = 16
_NW = _NC * _NS

_E = 320000
_D = 128
_N = 10000
_NP = 10240
_RPT = _NP // _NS

_C = 80
_S = 3
_CS = 128
_SS = 2

_ES0 = 163840
_ES1 = _E - _ES0

_sc_mesh = plsc.VectorSubcoreMesh(core_axis_name="c", subcore_axis_name="s")


def _make_gather(es):
    epw = es // _NW
    nchunk = epw // _C

    def body(hid_hbm, rela_hbm, qemb_hbm, idxc_hbm, rows_out,
             idx_v, rows_v, gsem, wsem):
        wid = lax.axis_index("s") * _NC + lax.axis_index("c")
        base0 = wid * epw
        tabs = (hid_hbm, rela_hbm, qemb_hbm)
        iw = nchunk * 3 * _C
        pltpu.sync_copy(idxc_hbm.at[pl.ds(wid * iw, iw)], idx_v)

        def step(k, carry):
            b = base0 + k * (_S * _C)
            gcps = [[pltpu.async_copy(
                tabs[t].at[idx_v.at[pl.ds((k * _S + s) * 3 * _C + t * _C, _C)]],
                rows_v.at[s, t], gsem) for t in range(3)]
                for s in range(_S)]
            wcps = []
            for s in range(_S):
                for cp in gcps[s]:
                    cp.wait()
                wcps += [pltpu.async_copy(rows_v.at[s, t],
                                          rows_out.at[t, pl.ds(b + s * _C, _C)],
                                          wsem) for t in range(3)]
            for cp in wcps:
                cp.wait()
            return carry

        def chunk_tail(i, carry):
            b = base0 + i * _C
            cps = [pltpu.async_copy(
                tabs[t].at[idx_v.at[pl.ds(i * 3 * _C + t * _C, _C)]],
                rows_v.at[0, t], gsem) for t in range(3)]
            for cp in cps:
                cp.wait()
            cps = [pltpu.async_copy(rows_v.at[0, t],
                                    rows_out.at[t, pl.ds(b, _C)], wsem)
                   for t in range(3)]
            for cp in cps:
                cp.wait()
            return carry

        lax.fori_loop(0, nchunk // _S, step, 0)
        lax.fori_loop((nchunk // _S) * _S, nchunk, chunk_tail, 0)

    return pl.kernel(
        body,
        out_type=jax.ShapeDtypeStruct((3, es, _D), jnp.float32),
        mesh=_sc_mesh,
        scratch_types=[
            pltpu.VMEM((nchunk * 3 * _C,), jnp.int32),
            pltpu.VMEM((_S, 3, _C, _D), jnp.float32),
            pltpu.SemaphoreType.DMA,
            pltpu.SemaphoreType.DMA,
        ],
    )


_gather_s0 = _make_gather(_ES0)
_gather_s1 = _make_gather(_ES1)


def _make_scatter(es):
    nchs = es // _CS
    full = nchs // _NW
    rem = nchs % _NW

    def body(msg_hbm, obj2d_hbm, zeros_hbm, out_hbm,
             obj_v, msg_v, acc_sh, isem, lsem, ssem):
        cid = lax.axis_index("c")
        sid = lax.axis_index("s")
        wid = sid * _NC + cid
        row0 = sid * _RPT
        trip = full + jnp.where(wid < rem, 1, 0)

        pltpu.sync_copy(zeros_hbm.at[pl.ds(row0, _RPT)],
                        acc_sh.at[pl.ds(row0, _RPT)])
        plsc.subcore_barrier()

        def chunk_tail(j, carry):
            cidx = wid + j * _NW
            pltpu.sync_copy(obj2d_hbm.at[cidx], obj_v.at[0])
            pltpu.sync_copy(msg_hbm.at[pl.ds(cidx * _CS, _CS)], msg_v.at[0])
            pltpu.sync_copy(msg_v.at[0], acc_sh.at[obj_v.at[0]], add=True)
            return carry

        def step(k, carry):
            lcps = []
            for s in range(_SS):
                cidx = wid + (k * _SS + s) * _NW
                lcps.append([
                    pltpu.async_copy(obj2d_hbm.at[cidx], obj_v.at[s], isem),
                    pltpu.async_copy(msg_hbm.at[pl.ds(cidx * _CS, _CS)],
                                     msg_v.at[s], lsem)])
            scps = []
            for s in range(_SS):
                for cp in lcps[s]:
                    cp.wait()
                scps.append(pltpu.async_copy(msg_v.at[s], acc_sh.at[obj_v.at[s]],
                                             ssem, add=True))
            for cp in scps:
                cp.wait()
            return carry

        lax.fori_loop(0, full // _SS, step, 0)
        lax.fori_loop((full // _SS) * _SS, trip, chunk_tail, 0)
        plsc.subcore_barrier()
        pltpu.sync_copy(acc_sh.at[pl.ds(row0, _RPT)],
                        out_hbm.at[pl.ds(cid * _NP + row0, _RPT)])

    return pl.kernel(
        body,
        out_type=jax.ShapeDtypeStruct((_NC * _NP, _D), jnp.float32),
        mesh=_sc_mesh,
        scratch_types=[
            pltpu.VMEM((_SS, _CS), jnp.int32),
            pltpu.VMEM((_SS, _CS, _D), jnp.float32),
            pltpu.VMEM_SHARED((_NP, _D), jnp.float32),
            pltpu.SemaphoreType.DMA,
            pltpu.SemaphoreType.DMA,
            pltpu.SemaphoreType.DMA,
        ],
    )


_scatter_s0 = _make_scatter(_ES0)
_scatter_s1 = _make_scatter(_ES1)


_BE = 4096


def _dense_body(rows_ref, Ws_ref, Wsb_ref, Wr_ref, Wq_ref,
                Wqr_ref, wa_ref, wab_ref, alpha_ref, msg_ref):
    hs = rows_ref[0]
    hr = rows_ref[1]
    hq = rows_ref[2]
    pre = (jnp.dot(hs, Ws_ref[...], preferred_element_type=jnp.float32)
           + jnp.dot(hr, Wr_ref[...], preferred_element_type=jnp.float32)
           + jnp.dot(hq, Wq_ref[...], preferred_element_type=jnp.float32)
           + jnp.dot(hr * hq, Wqr_ref[...], preferred_element_type=jnp.float32)
           + Wsb_ref[...])
    pre = jnp.maximum(pre, 0.0)
    z = jnp.dot(pre, wa_ref[...], preferred_element_type=jnp.float32) + wab_ref[...]
    av = jax.nn.sigmoid(z)
    alpha_ref[...] = av
    msg_ref[...] = av * (hs * hr)


def _dense_stage(rows, Ws_W, Ws_b, Wr_W, Wq_W, Wqr_W, wa_W, wa_b):
    es = rows.shape[1]
    a = Ws_W.shape[1]
    full = lambda s: pl.BlockSpec(s, lambda i: (0,) * len(s))
    return pl.pallas_call(
        _dense_body,
        out_shape=(jax.ShapeDtypeStruct((es, 1), jnp.float32),
                   jax.ShapeDtypeStruct((es, _D), jnp.float32)),
        grid=(pl.cdiv(es, _BE),),
        in_specs=[
            pl.BlockSpec((3, _BE, _D), lambda i: (0, i, 0)),
            full((_D, a)), full((1, a)), full((_D, a)), full((_D, a)),
            full((_D, a)), full((a, 1)), full((1, 1)),
        ],
        out_specs=(pl.BlockSpec((_BE, 1), lambda i: (i, 0)),
                   pl.BlockSpec((_BE, _D), lambda i: (i, 0))),
    )(rows, Ws_W, Ws_b.reshape(1, a), Wr_W, Wq_W, Wqr_W, wa_W,
      wa_b.reshape(1, 1))


def _final_body(p_ref, w_ref, o_ref):
    s = p_ref[0, 0]
    for k in range(1, 4):
        s = s + p_ref[k // 2, k % 2]
    o_ref[...] = jnp.dot(s, w_ref[...], preferred_element_type=jnp.float32)


def _final_stage(parts, Wh_W):
    bn = 1000
    return pl.pallas_call(
        _final_body,
        out_shape=jax.ShapeDtypeStruct((_N, _D), jnp.float32),
        grid=(_N // bn,),
        in_specs=[
            pl.BlockSpec((2, 2, bn, _D), lambda i: (0, 0, i, 0)),
            pl.BlockSpec((_D, _D), lambda i: (0, 0)),
        ],
        out_specs=pl.BlockSpec((bn, _D), lambda i: (i, 0)),
    )(parts, Wh_W)


def kernel(q_sub, q_rel, q_emb, rela_embed, hidden, edges, nodes,
           old_nodes_new_idx, Ws_W, Ws_b, Wr_W, Wq_W, Wqr_W, wa_W, wa_b, Wh_W):
    l1 = edges.shape[0]
    n1 = nodes.shape[0]
    ei = edges.astype(jnp.int32)
    obj = ei[:, 5]
    idx3 = jnp.stack([ei[:, 4], ei[:, 2], ei[:, 0]], axis=0)
    idxc = idx3.reshape(3, _E // _C, _C).transpose(1, 0, 2)
    zeros = jnp.zeros((_NP, _D), jnp.float32)
    obj2d = obj.reshape(_E // _CS, _CS)

    g0 = _ES0 // _C
    c0 = _ES0 // _CS
    idxc0 = idxc[:g0].reshape(-1)
    idxc1 = idxc[g0:].reshape(-1)

    rows0 = _gather_s0(hidden, rela_embed, q_emb, idxc0)
    alpha0, msg0 = _dense_stage(rows0, Ws_W, Ws_b, Wr_W, Wq_W, Wqr_W, wa_W, wa_b)
    rows1 = _gather_s1(hidden, rela_embed, q_emb, idxc1)
    alpha1, msg1 = _dense_stage(rows1, Ws_W, Ws_b, Wr_W, Wq_W, Wqr_W, wa_W, wa_b)
    parts0 = _scatter_s0(msg0, obj2d[:c0], zeros)
    parts1 = _scatter_s1(msg1, obj2d[c0:], zeros)

    alpha = jnp.concatenate([alpha0, alpha1], axis=0)
    parts = jnp.stack([parts0.reshape(_NC, _NP, _D),
                       parts1.reshape(_NC, _NP, _D)], axis=0)
    hidden_new = _final_stage(parts, Wh_W)

    nq = q_sub.shape[0]
    num_node = jnp.array([n1 * 1.0 / nq, n1 * 1.0 / nq], dtype=jnp.float32)
    num_edge = jnp.array([l1 * 1.0 / nq, l1 * 1.0 / nq], dtype=jnp.float32)
    return (num_node, num_edge, hidden_new, alpha, nodes, edges,
            old_nodes_new_idx)
